# Initial kernel scaffold; baseline (speedup 1.0000x reference)
import jax
import jax.numpy as jnp
from jax.experimental import pallas as pl


def kernel(inputs):
    raise NotImplementedError("write your pallas kernel here")


# speedup vs baseline: 4.8153x; 4.8153x over previous
import functools

import jax
import jax.numpy as jnp
import numpy as np
from jax.experimental import pallas as pl
from jax.experimental.pallas import tpu as pltpu

_NB = 64
_NSEG = _NB - 1
_NS = 20
_NOUT = _NSEG * _NS


@functools.lru_cache(maxsize=None)
def _spline_matrix() -> np.ndarray:
    n = _NB
    z1 = -2.0 + np.sqrt(3.0)
    R = np.eye(n, dtype=np.float64)
    powers = z1 ** np.arange(n, dtype=np.float64)

    qt0 = (powers @ R) / (1.0 - z1 ** n)
    QT = np.zeros((n, n), dtype=np.float64)
    QT[0] = qt0
    for i in range(1, n):
        QT[i] = z1 * QT[i - 1] + R[i]
    q0 = -(6.0 * z1 / (1.0 - z1 ** n)) * (powers @ QT)
    qtn = z1 * q0 - 6.0 * z1 * QT[n - 1]
    Q = np.zeros((n, n), dtype=np.float64)
    Q[0] = q0
    Q[n - 1] = qtn
    carry = qtn
    for i in range(n - 2, 0, -1):
        carry = z1 * carry - 6.0 * z1 * QT[i]
        Q[i] = carry

    s = np.linspace(0.0, 1.0, _NS)
    idx = (np.arange(_NSEG)[:, None] + np.arange(4)[None, :]) % _NSEG
    Qs = Q[idx]
    Q0, Q1, Q2, Q3 = Qs[:, 0], Qs[:, 1], Qs[:, 2], Qs[:, 3]
    c3 = -Q0 / 6.0 + Q1 / 2.0 - Q2 / 2.0 + Q3 / 6.0
    c2 = Q0 / 2.0 - Q1 + Q2 / 2.0
    c1 = -Q0 / 2.0 + Q2 / 2.0
    c0 = Q0 / 6.0 + 2.0 * Q1 / 3.0 + Q2 / 6.0
    curve = (c3[:, None] * (s ** 3)[None, :, None]
             + c2[:, None] * (s ** 2)[None, :, None]
             + c1[:, None] * s[None, :, None]
             + c0[:, None])
    return curve.reshape(_NOUT, n).T


@functools.lru_cache(maxsize=None)
def _interleaved_operator() -> np.ndarray:
    M = _spline_matrix()
    W = np.zeros((2 * _NB, 2 * _NOUT), dtype=np.float64)
    W[0::2, 0::2] = M
    W[1::2, 1::2] = M
    return W.astype(np.float32)


def _matmul_body(x_ref, w_ref, o_ref):
    o_ref[...] = jnp.dot(x_ref[...], w_ref[...],
                         preferred_element_type=jnp.float32)


def kernel(inputs):
    B = inputs.shape[0]
    x = inputs.reshape(B, 2 * _NB)
    W = jnp.asarray(_interleaved_operator())

    bm = 512
    out = pl.pallas_call(
        _matmul_body,
        grid=(B // bm,),
        in_specs=[
            pl.BlockSpec((bm, 2 * _NB), lambda i: (i, 0)),
            pl.BlockSpec((2 * _NB, 2 * _NOUT), lambda i: (0, 0)),
        ],
        out_specs=pl.BlockSpec((bm, 2 * _NOUT), lambda i: (i, 0)),
        out_shape=jax.ShapeDtypeStruct((B, 2 * _NOUT), jnp.float32),
    )(x, W)
    return out.reshape(B, _NOUT, 1, 2)


# speedup vs baseline: 4.8365x; 1.0044x over previous
import functools

import jax
import jax.numpy as jnp
import numpy as np
from jax.experimental import pallas as pl
from jax.experimental.pallas import tpu as pltpu

_NB = 64
_NSEG = _NB - 1
_NS = 20
_NOUT = _NSEG * _NS


@functools.lru_cache(maxsize=None)
def _spline_matrix() -> np.ndarray:
    n = _NB
    z1 = -2.0 + np.sqrt(3.0)
    R = np.eye(n, dtype=np.float64)
    powers = z1 ** np.arange(n, dtype=np.float64)

    qt0 = (powers @ R) / (1.0 - z1 ** n)
    QT = np.zeros((n, n), dtype=np.float64)
    QT[0] = qt0
    for i in range(1, n):
        QT[i] = z1 * QT[i - 1] + R[i]
    q0 = -(6.0 * z1 / (1.0 - z1 ** n)) * (powers @ QT)
    qtn = z1 * q0 - 6.0 * z1 * QT[n - 1]
    Q = np.zeros((n, n), dtype=np.float64)
    Q[0] = q0
    Q[n - 1] = qtn
    carry = qtn
    for i in range(n - 2, 0, -1):
        carry = z1 * carry - 6.0 * z1 * QT[i]
        Q[i] = carry

    s = np.linspace(0.0, 1.0, _NS)
    idx = (np.arange(_NSEG)[:, None] + np.arange(4)[None, :]) % _NSEG
    Qs = Q[idx]
    Q0, Q1, Q2, Q3 = Qs[:, 0], Qs[:, 1], Qs[:, 2], Qs[:, 3]
    c3 = -Q0 / 6.0 + Q1 / 2.0 - Q2 / 2.0 + Q3 / 6.0
    c2 = Q0 / 2.0 - Q1 + Q2 / 2.0
    c1 = -Q0 / 2.0 + Q2 / 2.0
    c0 = Q0 / 6.0 + 2.0 * Q1 / 3.0 + Q2 / 6.0
    curve = (c3[:, None] * (s ** 3)[None, :, None]
             + c2[:, None] * (s ** 2)[None, :, None]
             + c1[:, None] * s[None, :, None]
             + c0[:, None])
    return curve.reshape(_NOUT, n).T


@functools.lru_cache(maxsize=None)
def _interleaved_operator() -> np.ndarray:
    M = _spline_matrix()
    W = np.zeros((2 * _NB, 2 * _NOUT), dtype=np.float64)
    W[0::2, 0::2] = M
    W[1::2, 1::2] = M
    return W.astype(np.float32)


def _matmul_body(x_ref, w_ref, o_ref):
    xb = x_ref[...].astype(jnp.bfloat16)
    o_ref[...] = jnp.dot(xb, w_ref[...],
                         preferred_element_type=jnp.float32)


def kernel(inputs):
    B = inputs.shape[0]
    x = inputs.reshape(B, 2 * _NB)
    W = jnp.asarray(_interleaved_operator()).astype(jnp.bfloat16)

    bm = 512
    out = pl.pallas_call(
        _matmul_body,
        grid=(B // bm,),
        in_specs=[
            pl.BlockSpec((bm, 2 * _NB), lambda i: (i, 0)),
            pl.BlockSpec((2 * _NB, 2 * _NOUT), lambda i: (0, 0)),
        ],
        out_specs=pl.BlockSpec((bm, 2 * _NOUT), lambda i: (i, 0)),
        out_shape=jax.ShapeDtypeStruct((B, 2 * _NOUT), jnp.float32),
    )(x, W)
    return out.reshape(B, _NOUT, 1, 2)


# speedup vs baseline: 4.8487x; 1.0025x over previous
import functools

import jax
import jax.numpy as jnp
import numpy as np
from jax.experimental import pallas as pl
from jax.experimental.pallas import tpu as pltpu

_NB = 64
_NSEG = _NB - 1
_NS = 20
_NOUT = _NSEG * _NS


@functools.lru_cache(maxsize=None)
def _spline_matrix() -> np.ndarray:
    n = _NB
    z1 = -2.0 + np.sqrt(3.0)
    R = np.eye(n, dtype=np.float64)
    powers = z1 ** np.arange(n, dtype=np.float64)

    qt0 = (powers @ R) / (1.0 - z1 ** n)
    QT = np.zeros((n, n), dtype=np.float64)
    QT[0] = qt0
    for i in range(1, n):
        QT[i] = z1 * QT[i - 1] + R[i]
    q0 = -(6.0 * z1 / (1.0 - z1 ** n)) * (powers @ QT)
    qtn = z1 * q0 - 6.0 * z1 * QT[n - 1]
    Q = np.zeros((n, n), dtype=np.float64)
    Q[0] = q0
    Q[n - 1] = qtn
    carry = qtn
    for i in range(n - 2, 0, -1):
        carry = z1 * carry - 6.0 * z1 * QT[i]
        Q[i] = carry

    s = np.linspace(0.0, 1.0, _NS)
    idx = (np.arange(_NSEG)[:, None] + np.arange(4)[None, :]) % _NSEG
    Qs = Q[idx]
    Q0, Q1, Q2, Q3 = Qs[:, 0], Qs[:, 1], Qs[:, 2], Qs[:, 3]
    c3 = -Q0 / 6.0 + Q1 / 2.0 - Q2 / 2.0 + Q3 / 6.0
    c2 = Q0 / 2.0 - Q1 + Q2 / 2.0
    c1 = -Q0 / 2.0 + Q2 / 2.0
    c0 = Q0 / 6.0 + 2.0 * Q1 / 3.0 + Q2 / 6.0
    curve = (c3[:, None] * (s ** 3)[None, :, None]
             + c2[:, None] * (s ** 2)[None, :, None]
             + c1[:, None] * s[None, :, None]
             + c0[:, None])
    return curve.reshape(_NOUT, n).T


@functools.lru_cache(maxsize=None)
def _interleaved_operator() -> np.ndarray:
    M = _spline_matrix()
    W = np.zeros((2 * _NB, 2 * _NOUT), dtype=np.float64)
    W[0::2, 0::2] = M
    W[1::2, 1::2] = M
    return W.astype(np.float32)


def _matmul_body(x_ref, w_ref, o_ref):
    xb = x_ref[...].astype(jnp.bfloat16)
    o_ref[...] = jnp.dot(xb, w_ref[...],
                         preferred_element_type=jnp.float32)


def kernel(inputs):
    B = inputs.shape[0]
    x = inputs.reshape(B, 2 * _NB)
    W = jnp.asarray(_interleaved_operator()).astype(jnp.bfloat16)

    bm = 2048
    out = pl.pallas_call(
        _matmul_body,
        grid=(B // bm,),
        in_specs=[
            pl.BlockSpec((bm, 2 * _NB), lambda i: (i, 0)),
            pl.BlockSpec((2 * _NB, 2 * _NOUT), lambda i: (0, 0)),
        ],
        out_specs=pl.BlockSpec((bm, 2 * _NOUT), lambda i: (i, 0)),
        out_shape=jax.ShapeDtypeStruct((B, 2 * _NOUT), jnp.float32),
    )(x, W)
    return out.reshape(B, _NOUT, 1, 2)
